# initial kernel scaffold (unmeasured)
import jax
import jax.numpy as jnp
from jax import lax
from jax.experimental import pallas as pl
from jax.experimental.pallas import tpu as pltpu


def kernel(
    x,
):
    def body(*refs):
        pass

    out_shape = jax.ShapeDtypeStruct(..., jnp.float32)
    return pl.pallas_call(body, out_shape=out_shape)(...)


# baseline (device time: 99074 ns/iter reference)
import jax
import jax.numpy as jnp
from jax import lax
from jax.experimental import pallas as pl
from jax.experimental.pallas import tpu as pltpu

N_Y = 4


def kernel(x):
    m_per, n = x.shape
    n_out = n // N_Y
    m_out = m_per * N_Y

    def body(x_ref, out_ref, send_sems, recv_sems):
        my_x = lax.axis_index("x")
        my_y = lax.axis_index("y")
        my_z = lax.axis_index("z")

        barrier_sem = pltpu.get_barrier_semaphore()
        for d in range(1, N_Y):
            peer = lax.rem(my_y + d, N_Y)
            pl.semaphore_signal(
                barrier_sem, inc=1,
                device_id=(my_x, peer, my_z),
                device_id_type=pl.DeviceIdType.MESH,
            )
        pl.semaphore_wait(barrier_sem, N_Y - 1)

        out_ref[pl.ds(my_y * m_per, m_per), :] = (
            x_ref[:, pl.ds(my_y * n_out, n_out)]
        )

        sends = []
        for d in range(1, N_Y):
            tgt = lax.rem(my_y + d, N_Y)
            rdma = pltpu.make_async_remote_copy(
                src_ref=x_ref.at[:, pl.ds(tgt * n_out, n_out)],
                dst_ref=out_ref.at[pl.ds(my_y * m_per, m_per), :],
                send_sem=send_sems.at[d - 1],
                recv_sem=recv_sems.at[d - 1],
                device_id=(my_x, tgt, my_z),
                device_id_type=pl.DeviceIdType.MESH,
            )
            rdma.start()
            sends.append(rdma)

        for d in range(1, N_Y):
            src = lax.rem(my_y - d + N_Y, N_Y)
            recv = pltpu.make_async_remote_copy(
                src_ref=x_ref.at[:, pl.ds(0, n_out)],
                dst_ref=out_ref.at[pl.ds(src * m_per, m_per), :],
                send_sem=send_sems.at[d - 1],
                recv_sem=recv_sems.at[d - 1],
                device_id=(my_x, src, my_z),
                device_id_type=pl.DeviceIdType.MESH,
            )
            recv.wait_recv()

        for rdma in sends:
            rdma.wait_send()

    out_shape = jax.ShapeDtypeStruct((m_out, n_out), x.dtype)
    return pl.pallas_call(
        body,
        out_shape=out_shape,
        in_specs=[pl.BlockSpec(memory_space=pltpu.VMEM)],
        out_specs=pl.BlockSpec(memory_space=pltpu.VMEM),
        scratch_shapes=[
            pltpu.SemaphoreType.DMA((N_Y - 1,)),
            pltpu.SemaphoreType.DMA((N_Y - 1,)),
        ],
        compiler_params=pltpu.CompilerParams(collective_id=0),
    )(x)


# device time: 71099 ns/iter; 1.3935x vs baseline; 1.3935x over previous
import jax
import jax.numpy as jnp
from jax import lax
from jax.experimental import pallas as pl
from jax.experimental.pallas import tpu as pltpu

N_Y = 4


def kernel(x):
    m_per, n = x.shape
    n_out = n // N_Y
    m_out = m_per * N_Y
    m_half = m_per // 2

    def body(x_ref, out_ref, comm_y, comm_x, ys_sems, yr_sems, xs_sems, xr_sems):
        my_x = lax.axis_index("x")
        my_y = lax.axis_index("y")
        my_z = lax.axis_index("z")
        peer_x = 1 - my_x

        barrier_sem = pltpu.get_barrier_semaphore()
        for d in range(1, N_Y):
            peer = lax.rem(my_y + d, N_Y)
            pl.semaphore_signal(
                barrier_sem, inc=1,
                device_id=(my_x, peer, my_z),
                device_id_type=pl.DeviceIdType.MESH,
            )
        pl.semaphore_signal(
            barrier_sem, inc=1,
            device_id=(peer_x, my_y, my_z),
            device_id_type=pl.DeviceIdType.MESH,
        )
        pl.semaphore_wait(barrier_sem, N_Y)

        for d in range(1, N_Y):
            tgt = lax.rem(my_y + d, N_Y)
            for xv in (0, 1):
                @pl.when(my_x == xv)
                def _(d=d, tgt=tgt, xv=xv):
                    rdma = pltpu.make_async_remote_copy(
                        src_ref=x_ref.at[
                            pl.ds(xv * m_half, m_half),
                            pl.ds(tgt * n_out, n_out),
                        ],
                        dst_ref=comm_y.at[d - 1],
                        send_sem=ys_sems.at[d - 1],
                        recv_sem=yr_sems.at[d - 1],
                        device_id=(my_x, tgt, my_z),
                        device_id_type=pl.DeviceIdType.MESH,
                    )
                    rdma.start()

        out_ref[pl.ds(my_y * m_per, m_per), :] = (
            x_ref[:, pl.ds(my_y * n_out, n_out)]
        )

        def recv_wait(buf, slot, sems):
            dummy = pltpu.make_async_remote_copy(
                src_ref=buf.at[slot],
                dst_ref=buf.at[slot],
                send_sem=sems.at[slot],
                recv_sem=sems.at[slot],
                device_id=(my_x, my_y, my_z),
                device_id_type=pl.DeviceIdType.MESH,
            )
            dummy.wait_recv()

        for d in range(1, N_Y):
            src = lax.rem(my_y - d + N_Y, N_Y)
            recv_wait(comm_y, d - 1, yr_sems)
            fwd = pltpu.make_async_remote_copy(
                src_ref=comm_y.at[d - 1],
                dst_ref=comm_x.at[d - 1],
                send_sem=xs_sems.at[d - 1],
                recv_sem=xr_sems.at[d - 1],
                device_id=(peer_x, my_y, my_z),
                device_id_type=pl.DeviceIdType.MESH,
            )
            fwd.start()
            out_ref[pl.ds(src * m_per + my_x * m_half, m_half), :] = (
                comm_y[d - 1]
            )

        for d in range(1, N_Y):
            src = lax.rem(my_y - d + N_Y, N_Y)
            recv_wait(comm_x, d - 1, xr_sems)
            out_ref[pl.ds(src * m_per + peer_x * m_half, m_half), :] = (
                comm_x[d - 1]
            )

        def send_wait(buf, slot, sems):
            dummy = pltpu.make_async_remote_copy(
                src_ref=buf.at[slot],
                dst_ref=buf.at[slot],
                send_sem=sems.at[slot],
                recv_sem=sems.at[slot],
                device_id=(my_x, my_y, my_z),
                device_id_type=pl.DeviceIdType.MESH,
            )
            dummy.wait_send()

        for d in range(1, N_Y):
            send_wait(comm_y, d - 1, ys_sems)
            send_wait(comm_x, d - 1, xs_sems)

    out_shape = jax.ShapeDtypeStruct((m_out, n_out), x.dtype)
    return pl.pallas_call(
        body,
        out_shape=out_shape,
        in_specs=[pl.BlockSpec(memory_space=pltpu.VMEM)],
        out_specs=pl.BlockSpec(memory_space=pltpu.VMEM),
        scratch_shapes=[
            pltpu.VMEM((N_Y - 1, m_half, n_out), x.dtype),
            pltpu.VMEM((N_Y - 1, m_half, n_out), x.dtype),
            pltpu.SemaphoreType.DMA((N_Y - 1,)),
            pltpu.SemaphoreType.DMA((N_Y - 1,)),
            pltpu.SemaphoreType.DMA((N_Y - 1,)),
            pltpu.SemaphoreType.DMA((N_Y - 1,)),
        ],
        compiler_params=pltpu.CompilerParams(collective_id=0),
    )(x)
